# Initial kernel scaffold; baseline (speedup 1.0000x reference)
import functools

import jax
import jax.numpy as jnp
from jax import lax
from jax.experimental import pallas as pl
from jax.experimental.pallas import tpu as pltpu
from jax.experimental.pallas import tpu_sc as plsc

N = 10000
E = 160000
D = 256
K = 20

NC = 2
NS = 16
LANES = 16
DH = D // NC

CH = 128
NCHUNK = 80
EDGES_PER_TILE = NCHUNK * CH
E_PAD = NS * EDGES_PER_TILE
ROWS_PER_TILE = 640
N_PAD = NS * ROWS_PER_TILE
TRASH_SRC = N_PAD - 1
TRASH_DST = N
SUB = ROWS_PER_TILE // CH


def _rsqrt16(v):
    i = plsc.bitcast(v, jnp.int32)
    i = jnp.int32(0x5F3759DF) - (i >> 1)
    y = plsc.bitcast(i, jnp.float32)
    for _ in range(4):
        y = y * (1.5 - 0.5 * v * y * y)
    return y


def _sc_body(x_hbm, src_hbm, dst_hbm, u_hbm,
             acc, deg_o_sh, deg_i_sh,
             src_ref, dst_ref, gbuf, tbuf, zbuf, obuf, dbuf, rso_ref, srs_ref,
             sem):
    c = lax.axis_index("c")
    t = lax.axis_index("s")
    row0 = t * ROWS_PER_TILE
    zeros16 = jnp.zeros((LANES,), jnp.float32)
    ones16 = jnp.ones((LANES,), jnp.float32)

    pltpu.sync_copy(src_hbm.at[pl.ds(t * NCHUNK, NCHUNK)], src_ref)
    pltpu.sync_copy(dst_hbm.at[pl.ds(t * NCHUNK, NCHUNK)], dst_ref)

    @pl.loop(0, CH)
    def _(r):
        for g in range(CH // LANES):
            zbuf[r, pl.ds(g * LANES, LANES)] = zeros16
        obuf[r, pl.ds(0, LANES)] = ones16

    for s in range(SUB):
        base = row0 + s * CH
        pltpu.sync_copy(zbuf, acc.at[pl.ds(base, CH)])
        pltpu.sync_copy(zbuf.at[:, pl.ds(0, LANES)], deg_o_sh.at[pl.ds(base, CH)])
        pltpu.sync_copy(zbuf.at[:, pl.ds(0, LANES)], deg_i_sh.at[pl.ds(base, CH)])
    plsc.subcore_barrier()

    @pl.loop(0, NCHUNK)
    def _(j):
        pltpu.sync_copy(obuf, deg_o_sh.at[src_ref.at[j]], add=True)
        pltpu.sync_copy(obuf, deg_i_sh.at[dst_ref.at[j]], add=True)
    plsc.subcore_barrier()

    pltpu.sync_copy(deg_o_sh.at[pl.ds(row0, ROWS_PER_TILE)], dbuf)
    col0 = jnp.zeros((LANES,), jnp.int32)
    riota = lax.iota(jnp.int32, LANES)

    @pl.loop(0, ROWS_PER_TILE // LANES)
    def _(i):
        deg = plsc.load_gather(dbuf, [riota + i * LANES, col0])
        rso_ref[pl.ds(i * LANES, LANES)] = _rsqrt16(jnp.maximum(deg, 1.0))

    pltpu.sync_copy(deg_i_sh.at[pl.ds(row0, ROWS_PER_TILE)], dbuf)

    @pl.loop(0, ROWS_PER_TILE // LANES)
    def _(i):
        deg = plsc.load_gather(dbuf, [riota + i * LANES, col0])
        ri = _rsqrt16(jnp.maximum(deg, 1.0))
        ro = rso_ref[pl.ds(i * LANES, LANES)]
        srs_ref[0, pl.ds(i * LANES, LANES)] = ri * ro
        srs_ref[1, pl.ds(i * LANES, LANES)] = ri

    ubase = c * N_PAD

    @pl.loop(0, NCHUNK)
    def _(j):
        for g in range(CH // LANES):
            sl = pl.ds(g * LANES, LANES)
            src_ref[j, sl] = src_ref[j, sl] + ubase

    for s in range(SUB):
        base = row0 + s * CH
        pltpu.sync_copy(x_hbm.at[pl.ds(base, CH), pl.ds(c * DH, DH)], gbuf)

        @pl.loop(0, CH)
        def _(r):
            sc = rso_ref[s * CH + r]
            for g in range(DH // LANES):
                sl = pl.ds(g * LANES, LANES)
                gbuf[r, sl] = gbuf[r, sl] * sc

        pltpu.sync_copy(gbuf, u_hbm.at[pl.ds(ubase + base, CH)])
    plsc.subcore_barrier()

    def hop(srs_row):
        @pl.loop(0, NCHUNK)
        def _(j):
            pltpu.async_copy(u_hbm.at[src_ref.at[j]], gbuf, sem).wait()
            pltpu.sync_copy(gbuf, acc.at[dst_ref.at[j]], add=True)

        plsc.subcore_barrier()

        for s in range(SUB):
            base = row0 + s * CH
            pltpu.sync_copy(acc.at[pl.ds(base, CH)], tbuf)

            @pl.loop(0, CH)
            def _(r):
                sc = srs_ref[srs_row, s * CH + r]
                for g in range(DH // LANES):
                    sl = pl.ds(g * LANES, LANES)
                    tbuf[r, sl] = tbuf[r, sl] * sc

            pltpu.sync_copy(tbuf, u_hbm.at[pl.ds(ubase + base, CH)])
            pltpu.sync_copy(zbuf, acc.at[pl.ds(base, CH)])
        plsc.subcore_barrier()

    @pl.loop(0, K - 1)
    def _(k):
        hop(0)

    hop(1)


@functools.partial(
    pl.kernel,
    out_type=jax.ShapeDtypeStruct((NC * N_PAD, DH), jnp.float32),
    mesh=plsc.VectorSubcoreMesh(
        core_axis_name="c", subcore_axis_name="s", num_cores=NC, num_subcores=NS
    ),
    scratch_types=dict(
        acc=pltpu.VMEM_SHARED((N_PAD, DH), jnp.float32),
        deg_o_sh=pltpu.VMEM_SHARED((N_PAD, LANES), jnp.float32),
        deg_i_sh=pltpu.VMEM_SHARED((N_PAD, LANES), jnp.float32),
        src_ref=pltpu.VMEM((NCHUNK, CH), jnp.int32),
        dst_ref=pltpu.VMEM((NCHUNK, CH), jnp.int32),
        gbuf=pltpu.VMEM((CH, DH), jnp.float32),
        tbuf=pltpu.VMEM((CH, DH), jnp.float32),
        zbuf=pltpu.VMEM((CH, DH), jnp.float32),
        obuf=pltpu.VMEM((CH, LANES), jnp.float32),
        dbuf=pltpu.VMEM((ROWS_PER_TILE, LANES), jnp.float32),
        rso_ref=pltpu.VMEM((ROWS_PER_TILE,), jnp.float32),
        srs_ref=pltpu.VMEM((2, ROWS_PER_TILE), jnp.float32),
        sem=pltpu.SemaphoreType.DMA,
    ),
)
def _sc_propagate(x_hbm, src_hbm, dst_hbm, u_hbm, **scratch):
    _sc_body(x_hbm, src_hbm, dst_hbm, u_hbm, **scratch)


def _mm_body(h0_ref, h1_ref, w0_ref, w1_ref, b_ref, o_ref):
    o_ref[...] = (
        jnp.dot(h0_ref[...], w0_ref[...], preferred_element_type=jnp.float32)
        + jnp.dot(h1_ref[...], w1_ref[...], preferred_element_type=jnp.float32)
        + b_ref[...]
    )


_MM_BLOCK = 2000


def _tc_matmul(h0, h1, w0, w1, b2):
    return pl.pallas_call(
        _mm_body,
        grid=(N // _MM_BLOCK,),
        in_specs=[
            pl.BlockSpec((_MM_BLOCK, DH), lambda i: (i, 0)),
            pl.BlockSpec((_MM_BLOCK, DH), lambda i: (i, 0)),
            pl.BlockSpec((DH, D), lambda i: (0, 0)),
            pl.BlockSpec((DH, D), lambda i: (0, 0)),
            pl.BlockSpec((1, D), lambda i: (0, 0)),
        ],
        out_specs=pl.BlockSpec((_MM_BLOCK, D), lambda i: (i, 0)),
        out_shape=jax.ShapeDtypeStruct((N, D), jnp.float32),
    )(h0, h1, w0, w1, b2)


def kernel(x, edge_index, W, b):
    src = edge_index[0].astype(jnp.int32)
    dst = edge_index[1].astype(jnp.int32)
    pad = E_PAD - E
    src_p = jnp.concatenate([src, jnp.full((pad,), TRASH_SRC, jnp.int32)])
    dst_p = jnp.concatenate([dst, jnp.full((pad,), TRASH_DST, jnp.int32)])
    src2 = src_p.reshape(NS * NCHUNK, CH)
    dst2 = dst_p.reshape(NS * NCHUNK, CH)
    x_pad = jnp.pad(x, ((0, N_PAD - N), (0, 0)))

    u = _sc_propagate(x_pad, src2, dst2)

    h0 = u[0:N]
    h1 = u[N_PAD:N_PAD + N]
    return _tc_matmul(h0, h1, W[:DH], W[DH:], b.reshape(1, D))


# speedup vs baseline: 2.8028x; 2.8028x over previous
import functools

import jax
import jax.numpy as jnp
from jax import lax
from jax.experimental import pallas as pl
from jax.experimental.pallas import tpu as pltpu
from jax.experimental.pallas import tpu_sc as plsc

N = 10000
E = 160000
D = 256
K = 20

NC = 2
NS = 16
LANES = 16
DH = D // NC

CH = 128
NCHUNK = 80
NGRP = NCHUNK // 8
EDGES_PER_TILE = NCHUNK * CH
E_PAD = NS * EDGES_PER_TILE
ROWS_PER_TILE = 640
N_PAD = NS * ROWS_PER_TILE
TRASH_SRC = N_PAD - 1
TRASH_DST = N
SUB = ROWS_PER_TILE // CH
ZROW = N_PAD - CH


def _rsqrt16(v):
    y = 1.0 / v
    for _ in range(22):
        y = y * (1.5 - 0.5 * v * y * y)
    return y


def _sc_body(x_hbm, srcr_hbm, srco_hbm, dst_hbm, u_hbm,
             acc, sring, dring, gbuf, rso_p, srs_p, sem):
    c = lax.axis_index("c")
    t = lax.axis_index("s")
    row0 = t * ROWS_PER_TILE
    ones16 = jnp.ones((LANES,), jnp.float32)
    lanes_sl = pl.ds(0, LANES)

    def zero_acc(base):
        pltpu.sync_copy(x_hbm.at[pl.ds(ZROW, CH), pl.ds(c * DH, DH)],
                        acc.at[pl.ds(base, CH)])

    def fill_gbuf_ones():
        @pl.loop(0, CH)
        def _(r):
            for g in range(DH // LANES):
                gbuf[r, pl.ds(g * LANES, LANES)] = ones16

    for s in range(SUB):
        zero_acc(row0 + s * CH)
    plsc.subcore_barrier()

    def scatter_ones(idx_hbm):
        @pl.loop(0, NGRP)
        def _(g):
            pltpu.sync_copy(idx_hbm.at[t * NGRP + g], dring)

            @pl.loop(0, 8)
            def _(r):
                pltpu.sync_copy(gbuf, acc.at[dring.at[r]], add=True)

    def pack_degs(ii):
        liota = lax.iota(jnp.int32, LANES)
        dv = jnp.zeros((LANES,), jnp.float32)
        for lane in range(LANES):
            dv = jnp.where(liota == lane, gbuf[ii * LANES + lane, lanes_sl], dv)
        return dv

    fill_gbuf_ones()
    scatter_ones(srcr_hbm)
    plsc.subcore_barrier()

    for s in range(SUB):
        base = row0 + s * CH
        pltpu.sync_copy(acc.at[pl.ds(base, CH)], gbuf)

        @pl.loop(0, CH // LANES)
        def _(ii):
            rso_p[s, pl.ds(ii * LANES, LANES)] = _rsqrt16(
                jnp.maximum(pack_degs(ii), 1.0))

        zero_acc(base)
    plsc.subcore_barrier()

    fill_gbuf_ones()
    scatter_ones(dst_hbm)
    plsc.subcore_barrier()

    for s in range(SUB):
        base = row0 + s * CH
        pltpu.sync_copy(acc.at[pl.ds(base, CH)], gbuf)

        @pl.loop(0, CH // LANES)
        def _(ii):
            sl = pl.ds(ii * LANES, LANES)
            ri = _rsqrt16(jnp.maximum(pack_degs(ii), 1.0))
            srs_p[0, s, sl] = ri * rso_p[s, sl]
            srs_p[1, s, sl] = ri

        zero_acc(base)

    for s in range(SUB):
        base = row0 + s * CH
        pltpu.sync_copy(x_hbm.at[pl.ds(base, CH), pl.ds(c * DH, DH)], gbuf)

        @pl.loop(0, CH // LANES)
        def _(ii):
            svec = rso_p[s, pl.ds(ii * LANES, LANES)]
            for lane in range(LANES):
                sc = svec[lane]
                r = ii * LANES + lane
                for g in range(DH // LANES):
                    sl = pl.ds(g * LANES, LANES)
                    gbuf[r, sl] = gbuf[r, sl] * sc

        pltpu.sync_copy(gbuf, u_hbm.at[pl.ds(c * N_PAD + base, CH)])
    plsc.subcore_barrier()

    def hop(srs_row):
        @pl.loop(0, NGRP)
        def _(g):
            pltpu.sync_copy(srco_hbm.at[c, t * NGRP + g], sring)
            pltpu.sync_copy(dst_hbm.at[t * NGRP + g], dring)

            @pl.loop(0, 8)
            def _(r):
                pltpu.async_copy(u_hbm.at[sring.at[r]], gbuf, sem).wait()
                pltpu.sync_copy(gbuf, acc.at[dring.at[r]], add=True)

        plsc.subcore_barrier()

        for s in range(SUB):
            base = row0 + s * CH
            pltpu.sync_copy(acc.at[pl.ds(base, CH)], gbuf)

            @pl.loop(0, CH // LANES)
            def _(ii):
                svec = srs_p[srs_row, s, pl.ds(ii * LANES, LANES)]
                for lane in range(LANES):
                    sc = svec[lane]
                    r = ii * LANES + lane
                    for g in range(DH // LANES):
                        sl = pl.ds(g * LANES, LANES)
                        gbuf[r, sl] = gbuf[r, sl] * sc

            pltpu.sync_copy(gbuf, u_hbm.at[pl.ds(c * N_PAD + base, CH)])
            zero_acc(base)
        plsc.subcore_barrier()

    @pl.loop(0, K - 1)
    def _(k):
        hop(0)

    hop(1)


@functools.partial(
    pl.kernel,
    out_type=jax.ShapeDtypeStruct((NC * N_PAD, DH), jnp.float32),
    mesh=plsc.VectorSubcoreMesh(
        core_axis_name="c", subcore_axis_name="s", num_cores=NC, num_subcores=NS
    ),
    scratch_types=dict(
        acc=pltpu.VMEM_SHARED((N_PAD, DH), jnp.float32),
        sring=pltpu.VMEM((8, CH), jnp.int32),
        dring=pltpu.VMEM((8, CH), jnp.int32),
        gbuf=pltpu.VMEM((CH, DH), jnp.float32),
        rso_p=pltpu.VMEM((SUB, CH), jnp.float32),
        srs_p=pltpu.VMEM((2, SUB, CH), jnp.float32),
        sem=pltpu.SemaphoreType.DMA,
    ),
)
def _sc_propagate(x_hbm, srcr_hbm, srco_hbm, dst_hbm, u_hbm, **scratch):
    _sc_body(x_hbm, srcr_hbm, srco_hbm, dst_hbm, u_hbm, **scratch)


def _mm_body(h0_ref, h1_ref, w0_ref, w1_ref, b_ref, o_ref):
    o_ref[...] = (
        jnp.dot(h0_ref[...], w0_ref[...], preferred_element_type=jnp.float32)
        + jnp.dot(h1_ref[...], w1_ref[...], preferred_element_type=jnp.float32)
        + b_ref[...]
    )


_MM_BLOCK = 2000


def _tc_matmul(h0, h1, w0, w1, b2):
    return pl.pallas_call(
        _mm_body,
        grid=(N // _MM_BLOCK,),
        in_specs=[
            pl.BlockSpec((_MM_BLOCK, DH), lambda i: (i, 0)),
            pl.BlockSpec((_MM_BLOCK, DH), lambda i: (i, 0)),
            pl.BlockSpec((DH, D), lambda i: (0, 0)),
            pl.BlockSpec((DH, D), lambda i: (0, 0)),
            pl.BlockSpec((1, D), lambda i: (0, 0)),
        ],
        out_specs=pl.BlockSpec((_MM_BLOCK, D), lambda i: (i, 0)),
        out_shape=jax.ShapeDtypeStruct((N, D), jnp.float32),
    )(h0, h1, w0, w1, b2)


def kernel(x, edge_index, W, b):
    src = edge_index[0].astype(jnp.int32)
    dst = edge_index[1].astype(jnp.int32)
    pad = E_PAD - E
    src_p = jnp.concatenate([src, jnp.full((pad,), TRASH_SRC, jnp.int32)])
    dst_p = jnp.concatenate([dst, jnp.full((pad,), TRASH_DST, jnp.int32)])
    src_raw = src_p.reshape(NS * NGRP, 8, CH)
    src_off = src_raw[None] + (jnp.arange(NC, dtype=jnp.int32) * N_PAD)[
        :, None, None, None]
    dst2 = dst_p.reshape(NS * NGRP, 8, CH)
    x_pad = jnp.pad(x, ((0, N_PAD - N), (0, 0)))

    u = _sc_propagate(x_pad, src_raw, src_off, dst2)

    h0 = u[0:N]
    h1 = u[N_PAD:N_PAD + N]
    return _tc_matmul(h0, h1, W[:DH], W[DH:], b.reshape(1, D))


# speedup vs baseline: 3.4541x; 1.2324x over previous
import functools

import jax
import jax.numpy as jnp
from jax import lax
from jax.experimental import pallas as pl
from jax.experimental.pallas import tpu as pltpu
from jax.experimental.pallas import tpu_sc as plsc

N = 10000
E = 160000
D = 256
K = 20

NC = 2
NS = 16
LANES = 16
DH = D // NC

CH = 128
NCHUNK = 80
NGRP = NCHUNK // 8
EDGES_PER_TILE = NCHUNK * CH
E_PAD = NS * EDGES_PER_TILE
ROWS_PER_TILE = 640
N_PAD = NS * ROWS_PER_TILE
TRASH_SRC = N_PAD - 1
TRASH_DST = N
SUB = ROWS_PER_TILE // CH
ZROW = N_PAD - CH


def _rsqrt16(v):
    y = 1.0 / v
    for _ in range(22):
        y = y * (1.5 - 0.5 * v * y * y)
    return y


def _sc_body(x_hbm, srcr_hbm, srco_hbm, dst_hbm, u_hbm,
             acc, sring, dring, gbuf, gbuf2, rso_p, srs_p,
             sem, sem2, rs_sem, rd_sem):
    c = lax.axis_index("c")
    t = lax.axis_index("s")
    row0 = t * ROWS_PER_TILE
    ones16 = jnp.ones((LANES,), jnp.float32)
    lanes_sl = pl.ds(0, LANES)

    def zero_acc(base):
        pltpu.sync_copy(x_hbm.at[pl.ds(ZROW, CH), pl.ds(c * DH, DH)],
                        acc.at[pl.ds(base, CH)])

    def fill_gbuf_ones():
        @pl.loop(0, CH)
        def _(r):
            for g in range(DH // LANES):
                gbuf[r, pl.ds(g * LANES, LANES)] = ones16

    for s in range(SUB):
        zero_acc(row0 + s * CH)
    plsc.subcore_barrier()

    def scatter_ones(idx_hbm):
        @pl.loop(0, NGRP)
        def _(g):
            pltpu.sync_copy(idx_hbm.at[t * NGRP + g], dring.at[0])

            @pl.loop(0, 8)
            def _(r):
                pltpu.sync_copy(gbuf, acc.at[dring.at[0, r]], add=True)

    def pack_degs(ii):
        liota = lax.iota(jnp.int32, LANES)
        dv = jnp.zeros((LANES,), jnp.float32)
        for lane in range(LANES):
            dv = jnp.where(liota == lane, gbuf[ii * LANES + lane, lanes_sl], dv)
        return dv

    fill_gbuf_ones()
    scatter_ones(srcr_hbm)
    plsc.subcore_barrier()

    for s in range(SUB):
        base = row0 + s * CH
        pltpu.sync_copy(acc.at[pl.ds(base, CH)], gbuf)

        @pl.loop(0, CH // LANES)
        def _(ii):
            rso_p[s, pl.ds(ii * LANES, LANES)] = _rsqrt16(
                jnp.maximum(pack_degs(ii), 1.0))

        zero_acc(base)
    plsc.subcore_barrier()

    fill_gbuf_ones()
    scatter_ones(dst_hbm)
    plsc.subcore_barrier()

    for s in range(SUB):
        base = row0 + s * CH
        pltpu.sync_copy(acc.at[pl.ds(base, CH)], gbuf)

        @pl.loop(0, CH // LANES)
        def _(ii):
            sl = pl.ds(ii * LANES, LANES)
            ri = _rsqrt16(jnp.maximum(pack_degs(ii), 1.0))
            srs_p[0, s, sl] = ri * rso_p[s, sl]
            srs_p[1, s, sl] = ri

        zero_acc(base)

    for s in range(SUB):
        base = row0 + s * CH
        pltpu.sync_copy(x_hbm.at[pl.ds(base, CH), pl.ds(c * DH, DH)], gbuf)

        @pl.loop(0, CH // LANES)
        def _(ii):
            svec = rso_p[s, pl.ds(ii * LANES, LANES)]
            for lane in range(LANES):
                sc = svec[lane]
                r = ii * LANES + lane
                for g in range(DH // LANES):
                    sl = pl.ds(g * LANES, LANES)
                    gbuf[r, sl] = gbuf[r, sl] * sc

        pltpu.sync_copy(gbuf, u_hbm.at[pl.ds(c * N_PAD + base, CH)])
    plsc.subcore_barrier()

    bufs = (gbuf, gbuf2)
    sems = (sem, sem2)

    def hop(srs_row):
        pltpu.async_copy(srco_hbm.at[c, t * NGRP], sring.at[0], rs_sem).wait()
        pltpu.async_copy(dst_hbm.at[t * NGRP], dring.at[0], rd_sem).wait()
        pltpu.async_copy(u_hbm.at[sring.at[0, 0]], gbuf, sem)

        @pl.loop(0, NGRP)
        def _(g):
            p = g % 2

            @pl.when(g < NGRP - 1)
            def _():
                pltpu.async_copy(srco_hbm.at[c, t * NGRP + g + 1],
                                 sring.at[1 - p], rs_sem)
                pltpu.async_copy(dst_hbm.at[t * NGRP + g + 1],
                                 dring.at[1 - p], rd_sem)

            for r in range(8):
                buf, bsem = bufs[r % 2], sems[r % 2]
                nbuf, nsem = bufs[1 - r % 2], sems[1 - r % 2]
                if r < 7:
                    pltpu.async_copy(u_hbm.at[sring.at[p, r + 1]], nbuf, nsem)
                else:
                    @pl.when(g < NGRP - 1)
                    def _():
                        pltpu.make_async_copy(
                            srco_hbm.at[c, t * NGRP + g + 1],
                            sring.at[1 - p], rs_sem).wait()
                        pltpu.make_async_copy(
                            dst_hbm.at[t * NGRP + g + 1],
                            dring.at[1 - p], rd_sem).wait()
                        pltpu.async_copy(u_hbm.at[sring.at[1 - p, 0]],
                                         nbuf, nsem)
                pltpu.make_async_copy(u_hbm.at[pl.ds(0, CH)], buf, bsem).wait()
                pltpu.sync_copy(buf, acc.at[dring.at[p, r]], add=True)

        plsc.subcore_barrier()

        for s in range(SUB):
            base = row0 + s * CH
            pltpu.sync_copy(acc.at[pl.ds(base, CH)], gbuf)

            @pl.loop(0, CH // LANES)
            def _(ii):
                svec = srs_p[srs_row, s, pl.ds(ii * LANES, LANES)]
                for lane in range(LANES):
                    sc = svec[lane]
                    r = ii * LANES + lane
                    for g in range(DH // LANES):
                        sl = pl.ds(g * LANES, LANES)
                        gbuf[r, sl] = gbuf[r, sl] * sc

            pltpu.sync_copy(gbuf, u_hbm.at[pl.ds(c * N_PAD + base, CH)])
            zero_acc(base)
        plsc.subcore_barrier()

    @pl.loop(0, K - 1)
    def _(k):
        hop(0)

    hop(1)


@functools.partial(
    pl.kernel,
    out_type=jax.ShapeDtypeStruct((NC * N_PAD, DH), jnp.float32),
    mesh=plsc.VectorSubcoreMesh(
        core_axis_name="c", subcore_axis_name="s", num_cores=NC, num_subcores=NS
    ),
    scratch_types=dict(
        acc=pltpu.VMEM_SHARED((N_PAD, DH), jnp.float32),
        sring=pltpu.VMEM((2, 8, CH), jnp.int32),
        dring=pltpu.VMEM((2, 8, CH), jnp.int32),
        gbuf=pltpu.VMEM((CH, DH), jnp.float32),
        gbuf2=pltpu.VMEM((CH, DH), jnp.float32),
        rso_p=pltpu.VMEM((SUB, CH), jnp.float32),
        srs_p=pltpu.VMEM((2, SUB, CH), jnp.float32),
        sem=pltpu.SemaphoreType.DMA,
        sem2=pltpu.SemaphoreType.DMA,
        rs_sem=pltpu.SemaphoreType.DMA,
        rd_sem=pltpu.SemaphoreType.DMA,
    ),
)
def _sc_propagate(x_hbm, srcr_hbm, srco_hbm, dst_hbm, u_hbm, **scratch):
    _sc_body(x_hbm, srcr_hbm, srco_hbm, dst_hbm, u_hbm, **scratch)


def _mm_body(h0_ref, h1_ref, w0_ref, w1_ref, b_ref, o_ref):
    o_ref[...] = (
        jnp.dot(h0_ref[...], w0_ref[...], preferred_element_type=jnp.float32)
        + jnp.dot(h1_ref[...], w1_ref[...], preferred_element_type=jnp.float32)
        + b_ref[...]
    )


_MM_BLOCK = 2000


def _tc_matmul(h0, h1, w0, w1, b2):
    return pl.pallas_call(
        _mm_body,
        grid=(N // _MM_BLOCK,),
        in_specs=[
            pl.BlockSpec((_MM_BLOCK, DH), lambda i: (i, 0)),
            pl.BlockSpec((_MM_BLOCK, DH), lambda i: (i, 0)),
            pl.BlockSpec((DH, D), lambda i: (0, 0)),
            pl.BlockSpec((DH, D), lambda i: (0, 0)),
            pl.BlockSpec((1, D), lambda i: (0, 0)),
        ],
        out_specs=pl.BlockSpec((_MM_BLOCK, D), lambda i: (i, 0)),
        out_shape=jax.ShapeDtypeStruct((N, D), jnp.float32),
    )(h0, h1, w0, w1, b2)


def kernel(x, edge_index, W, b):
    src = edge_index[0].astype(jnp.int32)
    dst = edge_index[1].astype(jnp.int32)
    pad = E_PAD - E
    src_p = jnp.concatenate([src, jnp.full((pad,), TRASH_SRC, jnp.int32)])
    dst_p = jnp.concatenate([dst, jnp.full((pad,), TRASH_DST, jnp.int32)])
    src_raw = src_p.reshape(NS * NGRP, 8, CH)
    src_off = src_raw[None] + (jnp.arange(NC, dtype=jnp.int32) * N_PAD)[
        :, None, None, None]
    dst2 = dst_p.reshape(NS * NGRP, 8, CH)
    x_pad = jnp.pad(x, ((0, N_PAD - N), (0, 0)))

    u = _sc_propagate(x_pad, src_raw, src_off, dst2)

    h0 = u[0:N]
    h1 = u[N_PAD:N_PAD + N]
    return _tc_matmul(h0, h1, W[:DH], W[DH:], b.reshape(1, D))


# speedup vs baseline: 3.4546x; 1.0001x over previous
import functools

import jax
import jax.numpy as jnp
from jax import lax
from jax.experimental import pallas as pl
from jax.experimental.pallas import tpu as pltpu
from jax.experimental.pallas import tpu_sc as plsc

N = 10000
E = 160000
D = 256
K = 20

NC = 2
NS = 16
LANES = 16
DH = D // NC

CH = 128
NCHUNK = 80
NGRP = NCHUNK // 8
EDGES_PER_TILE = NCHUNK * CH
E_PAD = NS * EDGES_PER_TILE
ROWS_PER_TILE = 640
N_PAD = NS * ROWS_PER_TILE
TRASH_SRC = N_PAD - 1
TRASH_DST = N
SUB = ROWS_PER_TILE // CH
ZROW = N_PAD - CH


def _rsqrt16(v):
    y = 1.0 / v
    for _ in range(22):
        y = y * (1.5 - 0.5 * v * y * y)
    return y


def _sc_body(x_hbm, srcr_hbm, srco_hbm, dst_hbm, u_hbm,
             acc, sring, dring, gbuf, gbuf2, rso_p, srs_p,
             sem, sem2, rs_sem, rd_sem):
    c = lax.axis_index("c")
    t = lax.axis_index("s")
    row0 = t * ROWS_PER_TILE
    ones16 = jnp.ones((LANES,), jnp.float32)
    lanes_sl = pl.ds(0, LANES)

    def zero_acc(base):
        pltpu.sync_copy(x_hbm.at[pl.ds(ZROW, CH), pl.ds(c * DH, DH)],
                        acc.at[pl.ds(base, CH)])

    def fill_gbuf_ones():
        @pl.loop(0, CH)
        def _(r):
            for g in range(DH // LANES):
                gbuf[r, pl.ds(g * LANES, LANES)] = ones16

    for s in range(SUB):
        zero_acc(row0 + s * CH)
    plsc.subcore_barrier()

    def scatter_ones(idx_hbm):
        @pl.loop(0, NGRP)
        def _(g):
            pltpu.sync_copy(idx_hbm.at[t * NGRP + g], dring.at[0])

            @pl.loop(0, 8)
            def _(r):
                pltpu.sync_copy(gbuf, acc.at[dring.at[0, r]], add=True)

    def pack_degs(ii):
        liota = lax.iota(jnp.int32, LANES)
        dv = jnp.zeros((LANES,), jnp.float32)
        for lane in range(LANES):
            dv = jnp.where(liota == lane, gbuf[ii * LANES + lane, lanes_sl], dv)
        return dv

    fill_gbuf_ones()
    scatter_ones(srcr_hbm)
    plsc.subcore_barrier()

    for s in range(SUB):
        base = row0 + s * CH
        pltpu.sync_copy(acc.at[pl.ds(base, CH)], gbuf)

        @pl.loop(0, CH // LANES)
        def _(ii):
            rso_p[s, pl.ds(ii * LANES, LANES)] = _rsqrt16(
                jnp.maximum(pack_degs(ii), 1.0))

        zero_acc(base)
    plsc.subcore_barrier()

    fill_gbuf_ones()
    scatter_ones(dst_hbm)
    plsc.subcore_barrier()

    for s in range(SUB):
        base = row0 + s * CH
        pltpu.sync_copy(acc.at[pl.ds(base, CH)], gbuf)

        @pl.loop(0, CH // LANES)
        def _(ii):
            sl = pl.ds(ii * LANES, LANES)
            ri = _rsqrt16(jnp.maximum(pack_degs(ii), 1.0))
            srs_p[0, s, sl] = ri * rso_p[s, sl]
            srs_p[1, s, sl] = ri

        zero_acc(base)

    for s in range(SUB):
        base = row0 + s * CH
        pltpu.sync_copy(x_hbm.at[pl.ds(base, CH), pl.ds(c * DH, DH)], gbuf)

        @pl.loop(0, CH // LANES)
        def _(ii):
            svec = rso_p[s, pl.ds(ii * LANES, LANES)]
            for lane in range(LANES):
                sc = svec[lane]
                r = ii * LANES + lane
                for g in range(DH // LANES):
                    sl = pl.ds(g * LANES, LANES)
                    gbuf[r, sl] = gbuf[r, sl] * sc

        pltpu.sync_copy(gbuf, u_hbm.at[pl.ds(c * N_PAD + base, CH)])
    plsc.subcore_barrier()

    bufs = (gbuf, gbuf2)
    sems = (sem, sem2)

    def hop(srs_row):
        scope_gs = jax.named_scope("hop_gs")
        scope_gs.__enter__()
        pltpu.async_copy(srco_hbm.at[c, t * NGRP], sring.at[0], rs_sem).wait()
        pltpu.async_copy(dst_hbm.at[t * NGRP], dring.at[0], rd_sem).wait()
        pltpu.async_copy(u_hbm.at[sring.at[0, 0]], gbuf, sem)

        @pl.loop(0, NGRP)
        def _(g):
            p = g % 2

            @pl.when(g < NGRP - 1)
            def _():
                pltpu.async_copy(srco_hbm.at[c, t * NGRP + g + 1],
                                 sring.at[1 - p], rs_sem)
                pltpu.async_copy(dst_hbm.at[t * NGRP + g + 1],
                                 dring.at[1 - p], rd_sem)

            for r in range(8):
                buf, bsem = bufs[r % 2], sems[r % 2]
                nbuf, nsem = bufs[1 - r % 2], sems[1 - r % 2]
                if r < 7:
                    pltpu.async_copy(u_hbm.at[sring.at[p, r + 1]], nbuf, nsem)
                else:
                    @pl.when(g < NGRP - 1)
                    def _():
                        pltpu.make_async_copy(
                            srco_hbm.at[c, t * NGRP + g + 1],
                            sring.at[1 - p], rs_sem).wait()
                        pltpu.make_async_copy(
                            dst_hbm.at[t * NGRP + g + 1],
                            dring.at[1 - p], rd_sem).wait()
                        pltpu.async_copy(u_hbm.at[sring.at[1 - p, 0]],
                                         nbuf, nsem)
                pltpu.make_async_copy(u_hbm.at[pl.ds(0, CH)], buf, bsem).wait()
                pltpu.sync_copy(buf, acc.at[dring.at[p, r]], add=True)

        plsc.subcore_barrier()
        scope_gs.__exit__(None, None, None)

        scope_rb = jax.named_scope("hop_rb")
        scope_rb.__enter__()
        for s in range(SUB):
            base = row0 + s * CH
            pltpu.sync_copy(acc.at[pl.ds(base, CH)], gbuf)

            @pl.loop(0, CH // LANES)
            def _(ii):
                svec = srs_p[srs_row, s, pl.ds(ii * LANES, LANES)]
                for lane in range(LANES):
                    sc = svec[lane]
                    r = ii * LANES + lane
                    for g in range(DH // LANES):
                        sl = pl.ds(g * LANES, LANES)
                        gbuf[r, sl] = gbuf[r, sl] * sc

            pltpu.sync_copy(gbuf, u_hbm.at[pl.ds(c * N_PAD + base, CH)])
            zero_acc(base)
        plsc.subcore_barrier()
        scope_rb.__exit__(None, None, None)

    @pl.loop(0, K - 1)
    def _(k):
        hop(0)

    hop(1)


@functools.partial(
    pl.kernel,
    out_type=jax.ShapeDtypeStruct((NC * N_PAD, DH), jnp.float32),
    mesh=plsc.VectorSubcoreMesh(
        core_axis_name="c", subcore_axis_name="s", num_cores=NC, num_subcores=NS
    ),
    scratch_types=dict(
        acc=pltpu.VMEM_SHARED((N_PAD, DH), jnp.float32),
        sring=pltpu.VMEM((2, 8, CH), jnp.int32),
        dring=pltpu.VMEM((2, 8, CH), jnp.int32),
        gbuf=pltpu.VMEM((CH, DH), jnp.float32),
        gbuf2=pltpu.VMEM((CH, DH), jnp.float32),
        rso_p=pltpu.VMEM((SUB, CH), jnp.float32),
        srs_p=pltpu.VMEM((2, SUB, CH), jnp.float32),
        sem=pltpu.SemaphoreType.DMA,
        sem2=pltpu.SemaphoreType.DMA,
        rs_sem=pltpu.SemaphoreType.DMA,
        rd_sem=pltpu.SemaphoreType.DMA,
    ),
)
def _sc_propagate(x_hbm, srcr_hbm, srco_hbm, dst_hbm, u_hbm, **scratch):
    _sc_body(x_hbm, srcr_hbm, srco_hbm, dst_hbm, u_hbm, **scratch)


def _mm_body(h0_ref, h1_ref, w0_ref, w1_ref, b_ref, o_ref):
    o_ref[...] = (
        jnp.dot(h0_ref[...], w0_ref[...], preferred_element_type=jnp.float32)
        + jnp.dot(h1_ref[...], w1_ref[...], preferred_element_type=jnp.float32)
        + b_ref[...]
    )


_MM_BLOCK = 2000


def _tc_matmul(h0, h1, w0, w1, b2):
    return pl.pallas_call(
        _mm_body,
        grid=(N // _MM_BLOCK,),
        in_specs=[
            pl.BlockSpec((_MM_BLOCK, DH), lambda i: (i, 0)),
            pl.BlockSpec((_MM_BLOCK, DH), lambda i: (i, 0)),
            pl.BlockSpec((DH, D), lambda i: (0, 0)),
            pl.BlockSpec((DH, D), lambda i: (0, 0)),
            pl.BlockSpec((1, D), lambda i: (0, 0)),
        ],
        out_specs=pl.BlockSpec((_MM_BLOCK, D), lambda i: (i, 0)),
        out_shape=jax.ShapeDtypeStruct((N, D), jnp.float32),
    )(h0, h1, w0, w1, b2)


def kernel(x, edge_index, W, b):
    src = edge_index[0].astype(jnp.int32)
    dst = edge_index[1].astype(jnp.int32)
    pad = E_PAD - E
    src_p = jnp.concatenate([src, jnp.full((pad,), TRASH_SRC, jnp.int32)])
    dst_p = jnp.concatenate([dst, jnp.full((pad,), TRASH_DST, jnp.int32)])
    src_raw = src_p.reshape(NS * NGRP, 8, CH)
    src_off = src_raw[None] + (jnp.arange(NC, dtype=jnp.int32) * N_PAD)[
        :, None, None, None]
    dst2 = dst_p.reshape(NS * NGRP, 8, CH)
    x_pad = jnp.pad(x, ((0, N_PAD - N), (0, 0)))

    u = _sc_propagate(x_pad, src_raw, src_off, dst2)

    h0 = u[0:N]
    h1 = u[N_PAD:N_PAD + N]
    return _tc_matmul(h0, h1, W[:DH], W[DH:], b.reshape(1, D))


# speedup vs baseline: 4.3648x; 1.2635x over previous
import functools

import jax
import jax.numpy as jnp
from jax import lax
from jax.experimental import pallas as pl
from jax.experimental.pallas import tpu as pltpu
from jax.experimental.pallas import tpu_sc as plsc

N = 10000
E = 160000
D = 256
K = 20

NC = 2
NS = 16
LANES = 16
DH = D // NC

CH = 128
NCHUNK = 80
NGRP = NCHUNK // 8
E_PAD = NS * NCHUNK * CH
ROWS_PER_TILE = 640
N_PAD = NS * ROWS_PER_TILE
TRASH_SRC = N_PAD - 1
TRASH_DST = N
SUB = ROWS_PER_TILE // CH
ZROW = N_PAD - CH

WIN = 512
NBLK = N_PAD // WIN
WROWS = WIN // NS
G_CAP = E // 1024 + NBLK + 3


def _rsqrt16(v):
    y = 1.0 / v
    for _ in range(22):
        y = y * (1.5 - 0.5 * v * y * y)
    return y


def _sc_body(x_hbm, srcr_hbm, dstr_hbm, srcw_hbm, dstw_hbm, coff_hbm, u_hbm,
             acc, win, sring, dring, gbuf, gbuf2, rso_p, srs_p,
             coff_stage, coff_v, sem, sem2):
    c = lax.axis_index("c")
    t = lax.axis_index("s")
    row0 = t * ROWS_PER_TILE
    ones16 = jnp.ones((LANES,), jnp.float32)
    lanes_sl = pl.ds(0, LANES)
    liota = lax.iota(jnp.int32, LANES)

    pltpu.sync_copy(coff_hbm, coff_stage)
    for half in range(2):
        vh = coff_stage[pl.ds(half * LANES, LANES)]
        for lane in range(LANES):
            coff_v[half * LANES + lane] = vh[lane]

    def coff_at(i):
        return coff_v[i]

    def zero_acc(base, nrows=CH):
        pltpu.sync_copy(x_hbm.at[pl.ds(ZROW, nrows), pl.ds(c * DH, DH)],
                        acc.at[pl.ds(base, nrows)])

    def fill_gbuf_ones():
        @pl.loop(0, CH)
        def _(r):
            for g in range(DH // LANES):
                gbuf[r, pl.ds(g * LANES, LANES)] = ones16

    for s in range(SUB):
        zero_acc(row0 + s * CH)
    plsc.subcore_barrier()

    def scatter_ones(idx_hbm):
        @pl.loop(0, NGRP)
        def _(g):
            pltpu.sync_copy(idx_hbm.at[t * NGRP + g], dring)

            @pl.loop(0, 8)
            def _(r):
                pltpu.sync_copy(gbuf, acc.at[dring.at[r]], add=True)

    def pack_degs(ii):
        dv = jnp.zeros((LANES,), jnp.float32)
        for lane in range(LANES):
            dv = jnp.where(liota == lane, gbuf[ii * LANES + lane, lanes_sl], dv)
        return dv

    fill_gbuf_ones()
    scatter_ones(srcr_hbm)
    plsc.subcore_barrier()

    for s in range(SUB):
        base = row0 + s * CH
        pltpu.sync_copy(acc.at[pl.ds(base, CH)], gbuf)

        @pl.loop(0, CH // LANES)
        def _(ii):
            rso_p[s, pl.ds(ii * LANES, LANES)] = _rsqrt16(
                jnp.maximum(pack_degs(ii), 1.0))

        zero_acc(base)
    plsc.subcore_barrier()

    fill_gbuf_ones()
    scatter_ones(dstr_hbm)
    plsc.subcore_barrier()

    for s in range(SUB):
        base = row0 + s * CH
        pltpu.sync_copy(acc.at[pl.ds(base, CH)], gbuf)

        @pl.loop(0, CH // LANES)
        def _(ii):
            sl = pl.ds(ii * LANES, LANES)
            ri = _rsqrt16(jnp.maximum(pack_degs(ii), 1.0))
            srs_p[0, s, sl] = ri * rso_p[s, sl]
            srs_p[1, s, sl] = ri

        zero_acc(base)

    for s in range(SUB):
        base = row0 + s * CH
        pltpu.sync_copy(x_hbm.at[pl.ds(base, CH), pl.ds(c * DH, DH)], gbuf)

        @pl.loop(0, CH // LANES)
        def _(ii):
            svec = rso_p[s, pl.ds(ii * LANES, LANES)]
            for lane in range(LANES):
                sc = svec[lane]
                r = ii * LANES + lane
                for g in range(DH // LANES):
                    sl = pl.ds(g * LANES, LANES)
                    gbuf[r, sl] = gbuf[r, sl] * sc

        pltpu.sync_copy(gbuf, u_hbm.at[pl.ds(c * N_PAD + base, CH)])
    plsc.subcore_barrier()

    def issue_gather(rr, woff, dst_even):
        idx = sring.at[rr]

        @pl.when(dst_even)
        def _():
            pltpu.async_copy(win.at[idx], gbuf, sem)

        @pl.when(jnp.logical_not(dst_even))
        def _():
            pltpu.async_copy(win.at[idx], gbuf2, sem2)

    def hop(srs_row):
        pltpu.sync_copy(u_hbm.at[pl.ds(c * N_PAD + t * WROWS, WROWS)],
                        win.at[pl.ds(t * WROWS, WROWS)])

        @pl.loop(0, NBLK)
        def _(b):
            plsc.subcore_barrier()
            woff = (b % 2) * WIN

            @pl.when(b + 1 < NBLK)
            def _():
                nwoff = ((b + 1) % 2) * WIN
                pltpu.sync_copy(
                    u_hbm.at[pl.ds(c * N_PAD + (b + 1) * WIN + t * WROWS,
                                   WROWS)],
                    win.at[pl.ds(nwoff + t * WROWS, WROWS)])

            c0 = coff_at(b)
            c1 = coff_at(b + 1)
            nch = c1 - c0
            clo = c0 + (nch * t) // NS
            chi = c0 + (nch * (t + 1)) // NS

            @pl.loop(clo, chi)
            def _(ci):
                gi = ci // 8
                rr = ci % 8
                even = (rr % 2) == 0

                @pl.when((ci == clo) | (rr == 0))
                def _():
                    pltpu.sync_copy(srcw_hbm.at[gi], sring)
                    pltpu.sync_copy(dstw_hbm.at[gi], dring)

                    @pl.loop(0, 8)
                    def _(q):
                        for g in range(CH // LANES):
                            sl = pl.ds(g * LANES, LANES)
                            sring[q, sl] = sring[q, sl] + woff
                    issue_gather(rr, woff, even)

                @pl.when((rr < 7) & (ci + 1 < chi))
                def _():
                    issue_gather(rr + 1, woff, jnp.logical_not(even))

                @pl.when(even)
                def _():
                    pltpu.make_async_copy(
                        u_hbm.at[pl.ds(0, CH)], gbuf, sem).wait()
                    pltpu.sync_copy(gbuf, acc.at[dring.at[rr]], add=True)

                @pl.when(jnp.logical_not(even))
                def _():
                    pltpu.make_async_copy(
                        u_hbm.at[pl.ds(0, CH)], gbuf2, sem2).wait()
                    pltpu.sync_copy(gbuf2, acc.at[dring.at[rr]], add=True)

        plsc.subcore_barrier()

        for s in range(SUB):
            base = row0 + s * CH
            pltpu.sync_copy(acc.at[pl.ds(base, CH)], gbuf)

            @pl.loop(0, CH // LANES)
            def _(ii):
                svec = srs_p[srs_row, s, pl.ds(ii * LANES, LANES)]
                for lane in range(LANES):
                    sc = svec[lane]
                    r = ii * LANES + lane
                    for g in range(DH // LANES):
                        sl = pl.ds(g * LANES, LANES)
                        gbuf[r, sl] = gbuf[r, sl] * sc

            pltpu.sync_copy(gbuf, u_hbm.at[pl.ds(c * N_PAD + base, CH)])
            zero_acc(base)
        plsc.subcore_barrier()

    @pl.loop(0, K - 1)
    def _(k):
        hop(0)

    hop(1)


@functools.partial(
    pl.kernel,
    out_type=jax.ShapeDtypeStruct((NC * N_PAD, DH), jnp.float32),
    mesh=plsc.VectorSubcoreMesh(
        core_axis_name="c", subcore_axis_name="s", num_cores=NC, num_subcores=NS
    ),
    scratch_types=dict(
        acc=pltpu.VMEM_SHARED((N_PAD, DH), jnp.float32),
        win=pltpu.VMEM_SHARED((2 * WIN, DH), jnp.float32),
        sring=pltpu.VMEM((8, CH), jnp.int32),
        dring=pltpu.VMEM((8, CH), jnp.int32),
        gbuf=pltpu.VMEM((CH, DH), jnp.float32),
        gbuf2=pltpu.VMEM((CH, DH), jnp.float32),
        rso_p=pltpu.VMEM((SUB, CH), jnp.float32),
        srs_p=pltpu.VMEM((2, SUB, CH), jnp.float32),
        coff_stage=pltpu.VMEM((2 * LANES,), jnp.int32),
        coff_v=pltpu.SMEM((2 * LANES,), jnp.int32),
        sem=pltpu.SemaphoreType.DMA,
        sem2=pltpu.SemaphoreType.DMA,
    ),
)
def _sc_propagate(x_hbm, srcr_hbm, dstr_hbm, srcw_hbm, dstw_hbm, coff_hbm,
                  u_hbm, **scratch):
    _sc_body(x_hbm, srcr_hbm, dstr_hbm, srcw_hbm, dstw_hbm, coff_hbm, u_hbm,
             **scratch)


def _mm_body(h0_ref, h1_ref, w0_ref, w1_ref, b_ref, o_ref):
    o_ref[...] = (
        jnp.dot(h0_ref[...], w0_ref[...], preferred_element_type=jnp.float32)
        + jnp.dot(h1_ref[...], w1_ref[...], preferred_element_type=jnp.float32)
        + b_ref[...]
    )


_MM_BLOCK = 2000


def _tc_matmul(h0, h1, w0, w1, b2):
    return pl.pallas_call(
        _mm_body,
        grid=(N // _MM_BLOCK,),
        in_specs=[
            pl.BlockSpec((_MM_BLOCK, DH), lambda i: (i, 0)),
            pl.BlockSpec((_MM_BLOCK, DH), lambda i: (i, 0)),
            pl.BlockSpec((DH, D), lambda i: (0, 0)),
            pl.BlockSpec((DH, D), lambda i: (0, 0)),
            pl.BlockSpec((1, D), lambda i: (0, 0)),
        ],
        out_specs=pl.BlockSpec((_MM_BLOCK, D), lambda i: (i, 0)),
        out_shape=jax.ShapeDtypeStruct((N, D), jnp.float32),
    )(h0, h1, w0, w1, b2)


def kernel(x, edge_index, W, b):
    src = edge_index[0].astype(jnp.int32)
    dst = edge_index[1].astype(jnp.int32)

    pad = E_PAD - E
    src_raw = jnp.concatenate(
        [src, jnp.full((pad,), TRASH_SRC, jnp.int32)]).reshape(NS * NGRP, 8, CH)
    dst_raw = jnp.concatenate(
        [dst, jnp.full((pad,), TRASH_DST, jnp.int32)]).reshape(NS * NGRP, 8, CH)

    ordi = jnp.argsort(src)
    ss = src[ordi]
    dd = dst[ordi]
    eoff = jnp.searchsorted(
        ss, jnp.arange(NBLK + 1, dtype=jnp.int32) * WIN).astype(jnp.int32)
    ecnt = eoff[1:] - eoff[:-1]
    ggrp = (ecnt + 1023) // 1024
    goff = jnp.concatenate(
        [jnp.zeros((1,), jnp.int32), jnp.cumsum(ggrp).astype(jnp.int32)])
    j = jnp.arange(G_CAP * 1024, dtype=jnp.int32)
    jg = j // 1024
    blk = jnp.clip(
        jnp.searchsorted(goff, jg, side="right").astype(jnp.int32) - 1,
        0, NBLK - 1)
    o = j - goff[blk] * 1024
    valid = (o < ecnt[blk]) & (jg < goff[NBLK])
    e = jnp.clip(eoff[blk] + o, 0, E - 1)
    srcw = jnp.where(valid, ss[e] - blk * WIN, 0).reshape(G_CAP, 8, CH)
    dstw = jnp.where(valid, dd[e], TRASH_DST).reshape(G_CAP, 8, CH)
    coff = jnp.concatenate(
        [goff * 8, jnp.zeros((2 * LANES - NBLK - 1,), jnp.int32)])

    x_pad = jnp.pad(x, ((0, N_PAD - N), (0, 0)))

    u = _sc_propagate(x_pad, src_raw, dst_raw, srcw, dstw, coff)

    h0 = u[0:N]
    h1 = u[N_PAD:N_PAD + N]
    return _tc_matmul(h0, h1, W[:DH], W[DH:], b.reshape(1, D))


# speedup vs baseline: 4.3659x; 1.0003x over previous
import functools

import jax
import jax.numpy as jnp
from jax import lax
from jax.experimental import pallas as pl
from jax.experimental.pallas import tpu as pltpu
from jax.experimental.pallas import tpu_sc as plsc

N = 10000
E = 160000
D = 256
K = 20

NC = 2
NS = 16
LANES = 16
DH = D // NC

CH = 128
NCHUNK = 80
NGRP = NCHUNK // 8
E_PAD = NS * NCHUNK * CH
ROWS_PER_TILE = 640
N_PAD = NS * ROWS_PER_TILE
TRASH_SRC = N_PAD - 1
TRASH_DST = N
SUB = ROWS_PER_TILE // CH
ZROW = N_PAD - CH

WIN = 512
NBLK = N_PAD // WIN
WROWS = WIN // NS
G_CAP = E // 1024 + NBLK + 3


def _rsqrt16(v):
    y = 1.0 / v
    for _ in range(22):
        y = y * (1.5 - 0.5 * v * y * y)
    return y


def _sc_body(x_hbm, srcr_hbm, dstr_hbm, srcw_hbm, dstw_hbm, coff_hbm, u_hbm,
             acc, win, sring, dring, gbuf, gbuf2, rso_p, srs_p,
             coff_stage, coff_v, sem, sem2):
    c = lax.axis_index("c")
    t = lax.axis_index("s")
    row0 = t * ROWS_PER_TILE
    ones16 = jnp.ones((LANES,), jnp.float32)
    lanes_sl = pl.ds(0, LANES)
    liota = lax.iota(jnp.int32, LANES)

    pltpu.sync_copy(coff_hbm, coff_stage)
    for half in range(2):
        vh = coff_stage[pl.ds(half * LANES, LANES)]
        for lane in range(LANES):
            coff_v[half * LANES + lane] = vh[lane]

    def coff_at(i):
        return coff_v[i]

    def zero_acc(base, nrows=CH):
        pltpu.sync_copy(x_hbm.at[pl.ds(ZROW, nrows), pl.ds(c * DH, DH)],
                        acc.at[pl.ds(base, nrows)])

    def fill_gbuf_ones():
        @pl.loop(0, CH)
        def _(r):
            for g in range(DH // LANES):
                gbuf[r, pl.ds(g * LANES, LANES)] = ones16

    for s in range(SUB):
        zero_acc(row0 + s * CH)
    plsc.subcore_barrier()

    def scatter_ones(idx_hbm):
        @pl.loop(0, NGRP)
        def _(g):
            pltpu.sync_copy(idx_hbm.at[t * NGRP + g], dring)
            for r in range(8):
                pltpu.async_copy(gbuf, acc.at[dring.at[r]], sem, add=True)
            for r in range(8):
                pltpu.make_async_copy(gbuf, acc.at[pl.ds(0, CH)], sem).wait()

    def pack_degs(ii):
        dv = jnp.zeros((LANES,), jnp.float32)
        for lane in range(LANES):
            dv = jnp.where(liota == lane, gbuf[ii * LANES + lane, lanes_sl], dv)
        return dv

    fill_gbuf_ones()
    scatter_ones(srcr_hbm)
    plsc.subcore_barrier()

    for s in range(SUB):
        base = row0 + s * CH
        pltpu.sync_copy(acc.at[pl.ds(base, CH)], gbuf)

        @pl.loop(0, CH // LANES)
        def _(ii):
            rso_p[s, pl.ds(ii * LANES, LANES)] = _rsqrt16(
                jnp.maximum(pack_degs(ii), 1.0))

        zero_acc(base)
    plsc.subcore_barrier()

    fill_gbuf_ones()
    scatter_ones(dstr_hbm)
    plsc.subcore_barrier()

    for s in range(SUB):
        base = row0 + s * CH
        pltpu.sync_copy(acc.at[pl.ds(base, CH)], gbuf)

        @pl.loop(0, CH // LANES)
        def _(ii):
            sl = pl.ds(ii * LANES, LANES)
            ri = _rsqrt16(jnp.maximum(pack_degs(ii), 1.0))
            srs_p[0, s, sl] = ri * rso_p[s, sl]
            srs_p[1, s, sl] = ri

        zero_acc(base)

    for s in range(SUB):
        base = row0 + s * CH
        pltpu.sync_copy(x_hbm.at[pl.ds(base, CH), pl.ds(c * DH, DH)], gbuf)

        @pl.loop(0, CH // LANES)
        def _(ii):
            svec = rso_p[s, pl.ds(ii * LANES, LANES)]
            for lane in range(LANES):
                sc = svec[lane]
                r = ii * LANES + lane
                for g in range(DH // LANES):
                    sl = pl.ds(g * LANES, LANES)
                    gbuf[r, sl] = gbuf[r, sl] * sc

        pltpu.sync_copy(gbuf, u_hbm.at[pl.ds(c * N_PAD + base, CH)])
    plsc.subcore_barrier()

    def issue_gather(rr, woff, dst_even):
        idx = sring.at[rr]

        @pl.when(dst_even)
        def _():
            pltpu.async_copy(win.at[idx], gbuf, sem)

        @pl.when(jnp.logical_not(dst_even))
        def _():
            pltpu.async_copy(win.at[idx], gbuf2, sem2)

    def hop(srs_row):
        pltpu.sync_copy(u_hbm.at[pl.ds(c * N_PAD + t * WROWS, WROWS)],
                        win.at[pl.ds(t * WROWS, WROWS)])

        @pl.loop(0, NBLK)
        def _(b):
            plsc.subcore_barrier()
            woff = (b % 2) * WIN

            @pl.when(b + 1 < NBLK)
            def _():
                nwoff = ((b + 1) % 2) * WIN
                pltpu.sync_copy(
                    u_hbm.at[pl.ds(c * N_PAD + (b + 1) * WIN + t * WROWS,
                                   WROWS)],
                    win.at[pl.ds(nwoff + t * WROWS, WROWS)])

            c0 = coff_at(b)
            c1 = coff_at(b + 1)
            nch = c1 - c0
            clo = c0 + (nch * t) // NS
            chi = c0 + (nch * (t + 1)) // NS

            @pl.loop(clo, chi)
            def _(ci):
                gi = ci // 8
                rr = ci % 8
                even = (rr % 2) == 0

                @pl.when((ci == clo) | (rr == 0))
                def _():
                    pltpu.sync_copy(srcw_hbm.at[gi], sring)
                    pltpu.sync_copy(dstw_hbm.at[gi], dring)

                    @pl.loop(0, 8)
                    def _(q):
                        for g in range(CH // LANES):
                            sl = pl.ds(g * LANES, LANES)
                            sring[q, sl] = sring[q, sl] + woff
                    issue_gather(rr, woff, even)

                @pl.when((rr < 7) & (ci + 1 < chi))
                def _():
                    issue_gather(rr + 1, woff, jnp.logical_not(even))

                @pl.when(even)
                def _():
                    pltpu.make_async_copy(
                        u_hbm.at[pl.ds(0, CH)], gbuf, sem).wait()
                    pltpu.sync_copy(gbuf, acc.at[dring.at[rr]], add=True)

                @pl.when(jnp.logical_not(even))
                def _():
                    pltpu.make_async_copy(
                        u_hbm.at[pl.ds(0, CH)], gbuf2, sem2).wait()
                    pltpu.sync_copy(gbuf2, acc.at[dring.at[rr]], add=True)

        plsc.subcore_barrier()

        for s in range(SUB):
            base = row0 + s * CH
            pltpu.sync_copy(acc.at[pl.ds(base, CH)], gbuf)

            @pl.loop(0, CH // LANES)
            def _(ii):
                svec = srs_p[srs_row, s, pl.ds(ii * LANES, LANES)]
                for lane in range(LANES):
                    sc = svec[lane]
                    r = ii * LANES + lane
                    for g in range(DH // LANES):
                        sl = pl.ds(g * LANES, LANES)
                        gbuf[r, sl] = gbuf[r, sl] * sc

            pltpu.sync_copy(gbuf, u_hbm.at[pl.ds(c * N_PAD + base, CH)])
            zero_acc(base)
        plsc.subcore_barrier()

    @pl.loop(0, K - 1)
    def _(k):
        hop(0)

    hop(1)


@functools.partial(
    pl.kernel,
    out_type=jax.ShapeDtypeStruct((NC * N_PAD, DH), jnp.float32),
    mesh=plsc.VectorSubcoreMesh(
        core_axis_name="c", subcore_axis_name="s", num_cores=NC, num_subcores=NS
    ),
    scratch_types=dict(
        acc=pltpu.VMEM_SHARED((N_PAD, DH), jnp.float32),
        win=pltpu.VMEM_SHARED((2 * WIN, DH), jnp.float32),
        sring=pltpu.VMEM((8, CH), jnp.int32),
        dring=pltpu.VMEM((8, CH), jnp.int32),
        gbuf=pltpu.VMEM((CH, DH), jnp.float32),
        gbuf2=pltpu.VMEM((CH, DH), jnp.float32),
        rso_p=pltpu.VMEM((SUB, CH), jnp.float32),
        srs_p=pltpu.VMEM((2, SUB, CH), jnp.float32),
        coff_stage=pltpu.VMEM((2 * LANES,), jnp.int32),
        coff_v=pltpu.SMEM((2 * LANES,), jnp.int32),
        sem=pltpu.SemaphoreType.DMA,
        sem2=pltpu.SemaphoreType.DMA,
    ),
)
def _sc_propagate(x_hbm, srcr_hbm, dstr_hbm, srcw_hbm, dstw_hbm, coff_hbm,
                  u_hbm, **scratch):
    _sc_body(x_hbm, srcr_hbm, dstr_hbm, srcw_hbm, dstw_hbm, coff_hbm, u_hbm,
             **scratch)


def _mm_body(h0_ref, h1_ref, w0_ref, w1_ref, b_ref, o_ref):
    o_ref[...] = (
        jnp.dot(h0_ref[...], w0_ref[...], preferred_element_type=jnp.float32)
        + jnp.dot(h1_ref[...], w1_ref[...], preferred_element_type=jnp.float32)
        + b_ref[...]
    )


_MM_BLOCK = 2000


def _tc_matmul(h0, h1, w0, w1, b2):
    return pl.pallas_call(
        _mm_body,
        grid=(N // _MM_BLOCK,),
        in_specs=[
            pl.BlockSpec((_MM_BLOCK, DH), lambda i: (i, 0)),
            pl.BlockSpec((_MM_BLOCK, DH), lambda i: (i, 0)),
            pl.BlockSpec((DH, D), lambda i: (0, 0)),
            pl.BlockSpec((DH, D), lambda i: (0, 0)),
            pl.BlockSpec((1, D), lambda i: (0, 0)),
        ],
        out_specs=pl.BlockSpec((_MM_BLOCK, D), lambda i: (i, 0)),
        out_shape=jax.ShapeDtypeStruct((N, D), jnp.float32),
    )(h0, h1, w0, w1, b2)


def kernel(x, edge_index, W, b):
    src = edge_index[0].astype(jnp.int32)
    dst = edge_index[1].astype(jnp.int32)

    pad = E_PAD - E
    src_raw = jnp.concatenate(
        [src, jnp.full((pad,), TRASH_SRC, jnp.int32)]).reshape(NS * NGRP, 8, CH)
    dst_raw = jnp.concatenate(
        [dst, jnp.full((pad,), TRASH_DST, jnp.int32)]).reshape(NS * NGRP, 8, CH)

    ordi = jnp.argsort(src)
    ss = src[ordi]
    dd = dst[ordi]
    eoff = jnp.searchsorted(
        ss, jnp.arange(NBLK + 1, dtype=jnp.int32) * WIN).astype(jnp.int32)
    ecnt = eoff[1:] - eoff[:-1]
    ggrp = (ecnt + 1023) // 1024
    goff = jnp.concatenate(
        [jnp.zeros((1,), jnp.int32), jnp.cumsum(ggrp).astype(jnp.int32)])
    j = jnp.arange(G_CAP * 1024, dtype=jnp.int32)
    jg = j // 1024
    blk = jnp.clip(
        jnp.searchsorted(goff, jg, side="right").astype(jnp.int32) - 1,
        0, NBLK - 1)
    o = j - goff[blk] * 1024
    valid = (o < ecnt[blk]) & (jg < goff[NBLK])
    e = jnp.clip(eoff[blk] + o, 0, E - 1)
    srcw = jnp.where(valid, ss[e] - blk * WIN, 0).reshape(G_CAP, 8, CH)
    dstw = jnp.where(valid, dd[e], TRASH_DST).reshape(G_CAP, 8, CH)
    coff = jnp.concatenate(
        [goff * 8, jnp.zeros((2 * LANES - NBLK - 1,), jnp.int32)])

    x_pad = jnp.pad(x, ((0, N_PAD - N), (0, 0)))

    u = _sc_propagate(x_pad, src_raw, dst_raw, srcw, dstw, coff)

    h0 = u[0:N]
    h1 = u[N_PAD:N_PAD + N]
    return _tc_matmul(h0, h1, W[:DH], W[DH:], b.reshape(1, D))
